# Initial kernel scaffold; baseline (speedup 1.0000x reference)
import jax
import jax.numpy as jnp
from jax.experimental import pallas as pl


def kernel(data, u, W1, b1, W2, b2, code_book):
    raise NotImplementedError("write your pallas kernel here")


# speedup vs baseline: 1.2897x; 1.2897x over previous
import functools

import jax
import jax.numpy as jnp
from jax import lax
from jax.experimental import pallas as pl
from jax.experimental.pallas import tpu as pltpu
from jax.experimental.pallas import tpu_sc as plsc

G = 2
V = 1024
D = 512
DIN = 512
TAU = 0.5

TB1 = 256
TB3 = 512

SC_NC = 2
SC_NS = 16
SC_NW = SC_NC * SC_NS
SC_CHUNK = 128


def _stage1_body(x_ref, u_ref, w1_ref, b1_ref, p_ref, idx_ref):
    xb = x_ref[...]
    h = jnp.maximum(jnp.dot(xb, w1_ref[...]) + b1_ref[...], 0.0)
    xx = -jnp.log(-jnp.log(u_ref[...])) + h
    for g in range(G):
        xg = xx[:, g * V:(g + 1) * V]
        e = jnp.exp(xg)
        s = jnp.sum(e, axis=1, keepdims=True)
        p_ref[:, g * V:(g + 1) * V] = (e / s) * (1.0 / TAU)
        m = jnp.max(xg, axis=1, keepdims=True)
        io = lax.broadcasted_iota(jnp.int32, xg.shape, 1)
        idx = jnp.min(jnp.where(xg >= m, io, V), axis=1, keepdims=True)
        idx_ref[:, g:g + 1] = idx + g * V


def _stage1(x2d, u2d, w1p, b1p):
    bt = x2d.shape[0]
    grid = (bt // TB1,)
    return pl.pallas_call(
        _stage1_body,
        grid=grid,
        in_specs=[
            pl.BlockSpec((TB1, DIN), lambda i: (i, 0)),
            pl.BlockSpec((TB1, G * V), lambda i: (i, 0)),
            pl.BlockSpec((DIN, G * V), lambda i: (0, 0)),
            pl.BlockSpec((1, G * V), lambda i: (0, 0)),
        ],
        out_specs=[
            pl.BlockSpec((TB1, G * V), lambda i: (i, 0)),
            pl.BlockSpec((TB1, G), lambda i: (i, 0)),
        ],
        out_shape=[
            jax.ShapeDtypeStruct((bt, G * V), jnp.float32),
            jax.ShapeDtypeStruct((bt, G), jnp.int32),
        ],
    )(x2d, u2d, w1p, b1p)


def _stage3_body(s_ref, w2_ref, b2_ref, q_ref):
    q_ref[...] = jnp.maximum(jnp.dot(s_ref[...], w2_ref[...]) + b2_ref[...], 0.0)


def _stage3(sub, W2, b2row):
    bt = sub.shape[0]
    grid = (bt // TB3,)
    return pl.pallas_call(
        _stage3_body,
        grid=grid,
        in_specs=[
            pl.BlockSpec((TB3, G * D), lambda i: (i, 0)),
            pl.BlockSpec((G * D, D), lambda i: (0, 0)),
            pl.BlockSpec((1, D), lambda i: (0, 0)),
        ],
        out_specs=pl.BlockSpec((TB3, D), lambda i: (i, 0)),
        out_shape=jax.ShapeDtypeStruct((bt, D), jnp.float32),
    )(sub, W2, b2row)


def _sc_gather(table, idx3):
    nch = idx3.shape[1]
    rows_total = SC_NW * nch * SC_CHUNK
    mesh = plsc.VectorSubcoreMesh(core_axis_name="c", subcore_axis_name="s")

    @functools.partial(
        pl.kernel,
        mesh=mesh,
        out_type=jax.ShapeDtypeStruct((rows_total, D), jnp.float32),
        scratch_types=[
            pltpu.VMEM((nch, SC_CHUNK), jnp.int32),
            pltpu.VMEM((SC_CHUNK, D), jnp.float32),
            pltpu.SemaphoreType.DMA,
        ],
    )
    def k(table_hbm, idx_hbm, out_hbm, idx_v, rows_v, sem):
        wid = lax.axis_index("s") * SC_NC + lax.axis_index("c")
        pltpu.sync_copy(idx_hbm.at[wid], idx_v)
        base = wid * (nch * SC_CHUNK)
        for c in range(nch):
            pltpu.async_copy(table_hbm.at[idx_v.at[c]], rows_v, sem).wait()
            pltpu.sync_copy(rows_v, out_hbm.at[pl.ds(base + c * SC_CHUNK, SC_CHUNK)])

    return k(table, idx3)


def kernel(data, u, W1, b1, W2, b2, code_book):
    B, T, _ = data.shape
    bt = B * T
    x2d = data.reshape(bt, DIN)
    u2d = u.reshape(bt, G * V)
    w1p = W1.reshape(DIN, V, G).transpose(0, 2, 1).reshape(DIN, G * V)
    b1p = b1.reshape(V, G).T.reshape(1, G * V)

    p2d, idx2 = _stage1(x2d, u2d, w1p, b1p)

    nch = (bt * G) // (SC_NW * SC_CHUNK)
    idx3 = idx2.reshape(SC_NW, nch, SC_CHUNK)
    rows = _sc_gather(code_book.reshape(G * V, D), idx3)
    sub = rows.reshape(bt, G * D)

    q2d = _stage3(sub, W2, b2.reshape(1, D))
    return (p2d.reshape(B, T, G, V), q2d.reshape(B, T, D))


# speedup vs baseline: 1.8492x; 1.4338x over previous
import functools

import jax
import jax.numpy as jnp
from jax import lax
from jax.experimental import pallas as pl
from jax.experimental.pallas import tpu as pltpu
from jax.experimental.pallas import tpu_sc as plsc

G = 2
V = 1024
D = 512
DIN = 512
TAU = 0.5

TB1 = 256
TB3 = 512

SC_NC = 2
SC_NS = 16
SC_NW = SC_NC * SC_NS
SC_CHUNK = 128


def _stage1_body(x_ref, u_ref, w1_ref, b1_ref, p_ref, idx_ref):
    xb = x_ref[...]
    h = jnp.maximum(jnp.dot(xb, w1_ref[...]) + b1_ref[...], 0.0)
    idx_rows = []
    for g in range(G):
        xg = -jnp.log(-jnp.log(u_ref[:, g, :])) + h[:, g * V:(g + 1) * V]
        e = jnp.exp(xg)
        s = jnp.sum(e, axis=1, keepdims=True)
        p_ref[:, g, :] = (e / s) * (1.0 / TAU)
        m = jnp.max(xg, axis=1, keepdims=True)
        io = lax.broadcasted_iota(jnp.int32, xg.shape, 1)
        idx = jnp.min(jnp.where(xg >= m, io, V), axis=1, keepdims=True) + g * V
        idx_rows.append(jnp.reshape(idx, (TB1 // 128, 128)))
    idx_ref[0] = jnp.concatenate(idx_rows, axis=0)


def _stage1(x2d, u3d, w1p, b1p):
    bt = x2d.shape[0]
    grid = (bt // TB1,)
    return pl.pallas_call(
        _stage1_body,
        grid=grid,
        in_specs=[
            pl.BlockSpec((TB1, DIN), lambda i: (i, 0)),
            pl.BlockSpec((TB1, G, V), lambda i: (i, 0, 0)),
            pl.BlockSpec((DIN, G * V), lambda i: (0, 0)),
            pl.BlockSpec((1, G * V), lambda i: (0, 0)),
        ],
        out_specs=[
            pl.BlockSpec((TB1, G, V), lambda i: (i, 0, 0)),
            pl.BlockSpec((1, G * TB1 // 128, 128), lambda i: (i, 0, 0)),
        ],
        out_shape=[
            jax.ShapeDtypeStruct((bt, G, V), jnp.float32),
            jax.ShapeDtypeStruct((bt // TB1, G * TB1 // 128, 128), jnp.int32),
        ],
    )(x2d, u3d, w1p, b1p)


def _stage3_body(s_ref, w2_ref, b2_ref, q_ref):
    q_ref[...] = jnp.maximum(jnp.dot(s_ref[...], w2_ref[...]) + b2_ref[...], 0.0)


def _stage3(sub, W2, b2row):
    bt = sub.shape[0]
    grid = (bt // TB3,)
    return pl.pallas_call(
        _stage3_body,
        grid=grid,
        in_specs=[
            pl.BlockSpec((TB3, G * D), lambda i: (i, 0)),
            pl.BlockSpec((G * D, D), lambda i: (0, 0)),
            pl.BlockSpec((1, D), lambda i: (0, 0)),
        ],
        out_specs=pl.BlockSpec((TB3, D), lambda i: (i, 0)),
        out_shape=jax.ShapeDtypeStruct((bt, D), jnp.float32),
    )(sub, W2, b2row)


def _sc_gather(table, idx3, bt):
    rows_per_w = idx3.shape[1]
    tok_per_w = 2 * SC_CHUNK
    mesh = plsc.VectorSubcoreMesh(core_axis_name="c", subcore_axis_name="s")

    @functools.partial(
        pl.kernel,
        mesh=mesh,
        out_type=jax.ShapeDtypeStruct((bt, G * D), jnp.float32),
        scratch_types=[
            pltpu.VMEM((rows_per_w, SC_CHUNK), jnp.int32),
            pltpu.VMEM((SC_CHUNK, D), jnp.float32),
            pltpu.SemaphoreType.DMA,
        ],
    )
    def k(table_hbm, idx_hbm, out_hbm, idx_v, rows_v, sem):
        wid = lax.axis_index("s") * SC_NC + lax.axis_index("c")
        pltpu.sync_copy(idx_hbm.at[wid], idx_v)
        tok0 = wid * tok_per_w
        for g in range(G):
            for c in range(2):
                pltpu.async_copy(
                    table_hbm.at[idx_v.at[2 * g + c]], rows_v, sem).wait()
                pltpu.sync_copy(
                    rows_v,
                    out_hbm.at[pl.ds(tok0 + c * SC_CHUNK, SC_CHUNK),
                               pl.ds(g * D, D)])

    return k(table, idx3)


def kernel(data, u, W1, b1, W2, b2, code_book):
    B, T, _ = data.shape
    bt = B * T
    x2d = data.reshape(bt, DIN)
    u3d = u.reshape(bt, G, V)
    w1p = W1.reshape(DIN, V, G).transpose(0, 2, 1).reshape(DIN, G * V)
    b1p = b1.reshape(V, G).T.reshape(1, G * V)

    p3d, idx2 = _stage1(x2d, u3d, w1p, b1p)

    sub = _sc_gather(code_book.reshape(G * V, D), idx2, bt)

    q2d = _stage3(sub, W2, b2.reshape(1, D))
    return (p3d.reshape(B, T, G, V), q2d.reshape(B, T, D))


# speedup vs baseline: 1.9688x; 1.0647x over previous
import functools

import jax
import jax.numpy as jnp
from jax import lax
from jax.experimental import pallas as pl
from jax.experimental.pallas import tpu as pltpu
from jax.experimental.pallas import tpu_sc as plsc

G = 2
V = 1024
D = 512
DIN = 512
TAU = 0.5

TB1 = 256
TB3 = 512

SC_NC = 2
SC_NS = 16
SC_NW = SC_NC * SC_NS
SC_CHUNK = 128


def _stage1_body(*refs):
    x_ref, u_ref, w1_ref, b1_ref = refs[:4]
    p_ref, idx_ref = refs[-2:]
    xb = x_ref[...]
    h = jnp.maximum(jnp.dot(xb, w1_ref[...]) + b1_ref[...], 0.0)
    idx_rows = []
    for g in range(G):
        xg = -jnp.log(-jnp.log(u_ref[:, g, :])) + h[:, g * V:(g + 1) * V]
        e = jnp.exp(xg)
        s = jnp.sum(e, axis=1, keepdims=True)
        p_ref[:, g, :] = (e / s) * (1.0 / TAU)
        m = jnp.max(xg, axis=1, keepdims=True)
        io = lax.broadcasted_iota(jnp.int32, xg.shape, 1)
        idx = jnp.min(jnp.where(xg >= m, io, V), axis=1, keepdims=True) + g * V
        idx_rows.append(jnp.reshape(idx, (TB1 // 128, 128)))
    idx_ref[0] = jnp.concatenate(idx_rows, axis=0)


def _stage1(x2d, u3d, w1p, b1p, blk_lo, nblk, p_alias=None):
    bt = x2d.shape[0]
    in_specs = [
        pl.BlockSpec((TB1, DIN), lambda i, o=blk_lo: (i + o, 0)),
        pl.BlockSpec((TB1, G, V), lambda i, o=blk_lo: (i + o, 0, 0)),
        pl.BlockSpec((DIN, G * V), lambda i: (0, 0)),
        pl.BlockSpec((1, G * V), lambda i: (0, 0)),
    ]
    args = [x2d, u3d, w1p, b1p]
    aliases = {}
    if p_alias is not None:
        in_specs.append(pl.BlockSpec(memory_space=pl.ANY))
        args.append(p_alias)
        aliases = {4: 0}
    return pl.pallas_call(
        _stage1_body,
        grid=(nblk,),
        in_specs=in_specs,
        out_specs=[
            pl.BlockSpec((TB1, G, V), lambda i, o=blk_lo: (i + o, 0, 0)),
            pl.BlockSpec((1, G * TB1 // 128, 128), lambda i: (i, 0, 0)),
        ],
        out_shape=[
            jax.ShapeDtypeStruct((bt, G, V), jnp.float32),
            jax.ShapeDtypeStruct((nblk, G * TB1 // 128, 128), jnp.int32),
        ],
        input_output_aliases=aliases,
    )(*args)


def _stage3_body(*refs):
    s_ref, w2_ref, b2_ref = refs[:3]
    q_ref = refs[-1]
    q_ref[...] = jnp.maximum(jnp.dot(s_ref[...], w2_ref[...]) + b2_ref[...], 0.0)


def _stage3(sub, W2, b2row, bt, blk_lo, nblk, q_alias=None):
    in_specs = [
        pl.BlockSpec((TB3, G * D), lambda i: (i, 0)),
        pl.BlockSpec((G * D, D), lambda i: (0, 0)),
        pl.BlockSpec((1, D), lambda i: (0, 0)),
    ]
    args = [sub, W2, b2row]
    aliases = {}
    if q_alias is not None:
        in_specs.append(pl.BlockSpec(memory_space=pl.ANY))
        args.append(q_alias)
        aliases = {3: 0}
    return pl.pallas_call(
        _stage3_body,
        grid=(nblk,),
        in_specs=in_specs,
        out_specs=pl.BlockSpec((TB3, D), lambda i, o=blk_lo: (i + o, 0)),
        out_shape=jax.ShapeDtypeStruct((bt, D), jnp.float32),
        input_output_aliases=aliases,
    )(*args)


def _sc_gather(table, idx3, ow_lo, tok_half):
    mesh = plsc.VectorSubcoreMesh(core_axis_name="c", subcore_axis_name="s")

    @functools.partial(
        pl.kernel,
        mesh=mesh,
        out_type=jax.ShapeDtypeStruct((tok_half, G * D), jnp.float32),
        scratch_types=[
            pltpu.VMEM((2 * G, SC_CHUNK), jnp.int32),
            pltpu.VMEM((SC_CHUNK, D), jnp.float32),
            pltpu.SemaphoreType.DMA,
        ],
    )
    def k(table_hbm, idx_hbm, out_hbm, idx_v, rows_v, sem):
        wid = lax.axis_index("s") * SC_NC + lax.axis_index("c")
        ow = wid // 2
        c = wid % 2
        pltpu.sync_copy(idx_hbm.at[ow_lo + ow], idx_v)
        tok0 = ow * (2 * SC_CHUNK) + c * SC_CHUNK
        for g in range(G):
            pltpu.async_copy(table_hbm.at[idx_v.at[2 * g + c]], rows_v, sem).wait()
            pltpu.sync_copy(
                rows_v,
                out_hbm.at[pl.ds(tok0, SC_CHUNK), pl.ds(g * D, D)])

    return k(table, idx3)


def kernel(data, u, W1, b1, W2, b2, code_book):
    B, T, _ = data.shape
    bt = B * T
    half = bt // 2
    x2d = data.reshape(bt, DIN)
    u3d = u.reshape(bt, G, V)
    w1p = W1.reshape(DIN, V, G).transpose(0, 2, 1).reshape(DIN, G * V)
    b1p = b1.reshape(V, G).T.reshape(1, G * V)
    table = code_book.reshape(G * V, D)
    b2row = b2.reshape(1, D)

    nb1h = half // TB1
    nb3h = half // TB3

    p_a, idx_a = _stage1(x2d, u3d, w1p, b1p, 0, nb1h)
    p3d, idx_b = _stage1(x2d, u3d, w1p, b1p, nb1h, nb1h, p_alias=p_a)

    sub_a = _sc_gather(table, idx_a, 0, half)
    sub_b = _sc_gather(table, idx_b, 0, half)

    q_a = _stage3(sub_a, W2, b2row, bt, 0, nb3h)
    q2d = _stage3(sub_b, W2, b2row, bt, nb3h, nb3h, q_alias=q_a)

    return (p3d.reshape(B, T, G, V), q2d.reshape(B, T, D))
